# Initial kernel scaffold; baseline (speedup 1.0000x reference)
import functools

import jax
import jax.numpy as jnp
from jax import lax
from jax.experimental import pallas as pl
from jax.experimental.pallas import tpu as pltpu
from jax.experimental.pallas import tpu_sc as plsc

N_NODES = 10000
D = 128
E = 320000
NC = 2
NS = 16
NW = NC * NS
CHUNK = 80
EPW = E // NW
NCHUNK = EPW // CHUNK
ROWS_PER_TILE = N_NODES // NS
ZROWS = 125


def _sc_partial_sums(node, src, dst):
    mesh = plsc.VectorSubcoreMesh(core_axis_name="c", subcore_axis_name="s")

    @functools.partial(
        pl.kernel,
        out_type=jax.ShapeDtypeStruct((NC, N_NODES, D), jnp.float32),
        mesh=mesh,
        scratch_types=[
            pltpu.VMEM((NCHUNK, CHUNK), jnp.int32),
            pltpu.VMEM((NCHUNK, CHUNK), jnp.int32),
            pltpu.VMEM((CHUNK, D), jnp.float32),
            pltpu.VMEM((ZROWS, D), jnp.float32),
            pltpu.VMEM_SHARED((N_NODES, D), jnp.float32),
            pltpu.SemaphoreType.DMA,
        ],
    )
    def body(node_hbm, src_hbm, dst_hbm, out_hbm, sidx, didx, rows, zbuf, acc, sem):
        c = lax.axis_index("c")
        s = lax.axis_index("s")
        w = c * NS + s

        pltpu.sync_copy(src_hbm.at[pl.ds(w * NCHUNK, NCHUNK)], sidx)
        pltpu.sync_copy(dst_hbm.at[pl.ds(w * NCHUNK, NCHUNK)], didx)

        @pl.loop(0, ZROWS)
        def _(r):
            for j in range(D // 16):
                zbuf[r, pl.ds(j * 16, 16)] = jnp.zeros((16,), jnp.float32)

        for k in range(ROWS_PER_TILE // ZROWS):
            pltpu.sync_copy(
                zbuf, acc.at[pl.ds(s * ROWS_PER_TILE + k * ZROWS, ZROWS)])
        plsc.subcore_barrier()

        @pl.loop(0, NCHUNK)
        def _(i):
            pltpu.async_copy(node_hbm.at[sidx.at[i]], rows, sem).wait()
            pltpu.sync_copy(rows, acc.at[didx.at[i]], add=True)

        plsc.subcore_barrier()
        pltpu.sync_copy(
            acc.at[pl.ds(s * ROWS_PER_TILE, ROWS_PER_TILE)],
            out_hbm.at[c, pl.ds(s * ROWS_PER_TILE, ROWS_PER_TILE)])

    return body(node, src, dst)


def _combine(partials):
    def body(p_ref, o_ref):
        o_ref[...] = p_ref[0] + p_ref[1]

    return pl.pallas_call(
        body,
        out_shape=jax.ShapeDtypeStruct((N_NODES, D), jnp.float32),
    )(partials)


@jax.jit
def kernel(node, edge_index):
    ei = edge_index.astype(jnp.int32)
    src = ei[0].reshape(E // CHUNK, CHUNK)
    dst = ei[1].reshape(E // CHUNK, CHUNK)
    partials = _sc_partial_sums(node, src, dst)
    return _combine(partials)


# speedup vs baseline: 7.6838x; 7.6838x over previous
import functools

import jax
import jax.numpy as jnp
from jax import lax
from jax.experimental import pallas as pl
from jax.experimental.pallas import tpu as pltpu
from jax.experimental.pallas import tpu_sc as plsc

N_NODES = 10000
D = 128
E = 320000
NC = 2
NS = 16
NW = NC * NS
CHUNK = 80
EPW = E // NW
NCHUNK = EPW // CHUNK
NPAD = 10240
ROWS_PER_TILE = NPAD // NS
ZROWS = 16


def _sc_partial_sums(node, src, dst):
    mesh = plsc.VectorSubcoreMesh(core_axis_name="c", subcore_axis_name="s")

    @functools.partial(
        pl.kernel,
        out_type=jax.ShapeDtypeStruct((NC, NPAD, D), jnp.float32),
        mesh=mesh,
        scratch_types=[
            pltpu.VMEM((NCHUNK, CHUNK), jnp.int32),
            pltpu.VMEM((NCHUNK, CHUNK), jnp.int32),
            pltpu.VMEM((CHUNK, D), jnp.float32),
            pltpu.VMEM((ZROWS, D), jnp.float32),
            pltpu.VMEM_SHARED((NPAD, D), jnp.float32),
            pltpu.SemaphoreType.DMA,
        ],
    )
    def body(node_hbm, src_hbm, dst_hbm, out_hbm, sidx, didx, rows, zbuf, acc, sem):
        c = lax.axis_index("c")
        s = lax.axis_index("s")
        w = c * NS + s

        pltpu.sync_copy(src_hbm.at[w], sidx)
        pltpu.sync_copy(dst_hbm.at[w], didx)

        @pl.loop(0, ZROWS)
        def _(r):
            for j in range(D // 16):
                zbuf[r, pl.ds(j * 16, 16)] = jnp.zeros((16,), jnp.float32)

        for k in range(ROWS_PER_TILE // ZROWS):
            pltpu.sync_copy(
                zbuf, acc.at[pl.ds(s * ROWS_PER_TILE + k * ZROWS, ZROWS)])
        plsc.subcore_barrier()

        @pl.loop(0, NCHUNK)
        def _(i):
            pltpu.async_copy(node_hbm.at[sidx.at[i]], rows, sem).wait()
            pltpu.sync_copy(rows, acc.at[didx.at[i]], add=True)

        plsc.subcore_barrier()
        pltpu.sync_copy(
            acc.at[pl.ds(s * ROWS_PER_TILE, ROWS_PER_TILE)],
            out_hbm.at[c, pl.ds(s * ROWS_PER_TILE, ROWS_PER_TILE)])

    return body(node, src, dst)


def _combine(partials):
    def body(p_ref, o_ref):
        o_ref[...] = p_ref[0] + p_ref[1]

    return pl.pallas_call(
        body,
        out_shape=jax.ShapeDtypeStruct((NPAD, D), jnp.float32),
    )(partials)


@jax.jit
def kernel(node, edge_index):
    ei = edge_index.astype(jnp.int32)
    src = ei[0].reshape(NW, NCHUNK, CHUNK)
    dst = ei[1].reshape(NW, NCHUNK, CHUNK)
    partials = _sc_partial_sums(node, src, dst)
    return _combine(partials)[:N_NODES]


# speedup vs baseline: 8.2696x; 1.0762x over previous
import functools

import jax
import jax.numpy as jnp
from jax import lax
from jax.experimental import pallas as pl
from jax.experimental.pallas import tpu as pltpu
from jax.experimental.pallas import tpu_sc as plsc

N_NODES = 10000
D = 128
E = 320000
NC = 2
NS = 16
NW = NC * NS
CHUNK = 80
EPW = E // NW
NCHUNK = EPW // CHUNK
NPAD = 10240
ROWS_PER_TILE = NPAD // NS


def _sc_partial_sums(node, src, dst):
    mesh = plsc.VectorSubcoreMesh(core_axis_name="c", subcore_axis_name="s")

    @functools.partial(
        pl.kernel,
        out_type=jax.ShapeDtypeStruct((NC, NPAD, D), jnp.float32),
        mesh=mesh,
        scratch_types=[
            pltpu.VMEM((CHUNK, D), jnp.float32),
            pltpu.VMEM((CHUNK, D), jnp.float32),
            pltpu.VMEM((CHUNK,), jnp.int32),
            pltpu.VMEM((CHUNK,), jnp.int32),
            pltpu.VMEM((CHUNK,), jnp.int32),
            pltpu.VMEM((CHUNK,), jnp.int32),
            pltpu.VMEM_SHARED((NPAD, D), jnp.float32),
            pltpu.SemaphoreType.DMA,
            pltpu.SemaphoreType.DMA,
            pltpu.SemaphoreType.DMA,
            pltpu.SemaphoreType.DMA,
        ],
    )
    def body(node_hbm, src_hbm, dst_hbm, out_hbm,
             rows0, rows1, sx0, sx1, dx0, dx1, acc,
             g0, g1, s0, s1):
        rows = (rows0, rows1)
        sx = (sx0, sx1)
        dx = (dx0, dx1)
        gsem = (g0, g1)
        ssem = (s0, s1)

        c = lax.axis_index("c")
        s = lax.axis_index("s")
        w = c * NS + s

        @pl.loop(0, CHUNK)
        def _(r):
            for j in range(D // 16):
                rows0[r, pl.ds(j * 16, 16)] = jnp.zeros((16,), jnp.float32)

        for k in range(ROWS_PER_TILE // CHUNK):
            pltpu.sync_copy(
                rows0, acc.at[pl.ds(s * ROWS_PER_TILE + k * CHUNK, CHUNK)])
        plsc.subcore_barrier()

        def stage_and_gather(i, b):
            pltpu.sync_copy(src_hbm.at[w, i], sx[b])
            pltpu.sync_copy(dst_hbm.at[w, i], dx[b])
            pltpu.async_copy(node_hbm.at[sx[b]], rows[b], gsem[b])

        stage_and_gather(0, 0)
        stage_and_gather(1, 1)

        @pl.loop(0, (NCHUNK - 1) // 2)
        def _(t):
            for b in range(2):
                i = 2 * t + b
                pltpu.make_async_copy(node_hbm.at[sx[b]], rows[b],
                                      gsem[b]).wait()
                pltpu.async_copy(rows[b], acc.at[dx[b]], ssem[b], add=True)
                pltpu.make_async_copy(rows[b], acc.at[dx[b]], ssem[b]).wait()

                @pl.when(i + 2 < NCHUNK)
                def _():
                    stage_and_gather(i + 2, b)

        pltpu.make_async_copy(node_hbm.at[sx[0]], rows[0], gsem[0]).wait()
        pltpu.async_copy(rows[0], acc.at[dx[0]], ssem[0], add=True)
        pltpu.make_async_copy(rows[0], acc.at[dx[0]], ssem[0]).wait()

        plsc.subcore_barrier()
        pltpu.sync_copy(
            acc.at[pl.ds(s * ROWS_PER_TILE, ROWS_PER_TILE)],
            out_hbm.at[c, pl.ds(s * ROWS_PER_TILE, ROWS_PER_TILE)])

    return body(node, src, dst)


def _combine(partials):
    def body(p_ref, o_ref):
        o_ref[...] = p_ref[0] + p_ref[1]

    return pl.pallas_call(
        body,
        out_shape=jax.ShapeDtypeStruct((NPAD, D), jnp.float32),
    )(partials)


@jax.jit
def kernel(node, edge_index):
    ei = edge_index.astype(jnp.int32)
    src = ei[0].reshape(NW, NCHUNK, CHUNK)
    dst = ei[1].reshape(NW, NCHUNK, CHUNK)
    partials = _sc_partial_sums(node, src, dst)
    return _combine(partials)[:N_NODES]


# speedup vs baseline: 12.6602x; 1.5309x over previous
import functools

import jax
import jax.numpy as jnp
from jax import lax
from jax.experimental import pallas as pl
from jax.experimental.pallas import tpu as pltpu
from jax.experimental.pallas import tpu_sc as plsc

N_NODES = 10000
D = 128
E = 320000
NC = 2
NS = 16
NW = NC * NS
EPW = E // NW
CHUNK = 128
NFULL = EPW // CHUNK
TAIL = EPW - NFULL * CHUNK
NPAD = 10240
ROWS_PER_TILE = NPAD // NS


def _sc_partial_sums(node, eidx):
    mesh = plsc.VectorSubcoreMesh(core_axis_name="c", subcore_axis_name="s")

    @functools.partial(
        pl.kernel,
        out_type=jax.ShapeDtypeStruct((NC, NPAD, D), jnp.float32),
        mesh=mesh,
        scratch_types=[
            pltpu.VMEM((CHUNK, D), jnp.float32),
            pltpu.VMEM((CHUNK, D), jnp.float32),
            pltpu.VMEM((2, CHUNK), jnp.int32),
            pltpu.VMEM((2, CHUNK), jnp.int32),
            pltpu.VMEM((2, TAIL), jnp.int32),
            pltpu.VMEM_SHARED((NPAD, D), jnp.float32),
            pltpu.SemaphoreType.DMA,
            pltpu.SemaphoreType.DMA,
            pltpu.SemaphoreType.DMA,
            pltpu.SemaphoreType.DMA,
        ],
    )
    def body(node_hbm, eidx_hbm, out_hbm,
             rows0, rows1, ix0, ix1, ixt, acc,
             g0, g1, s0, s1):
        rows = (rows0, rows1)
        ix = (ix0, ix1)
        gsem = (g0, g1)
        ssem = (s0, s1)

        c = lax.axis_index("c")
        s = lax.axis_index("s")
        w = c * NS + s

        @pl.loop(0, CHUNK)
        def _(r):
            for j in range(D // 16):
                rows0[r, pl.ds(j * 16, 16)] = jnp.zeros((16,), jnp.float32)

        for k in range(ROWS_PER_TILE // CHUNK):
            pltpu.sync_copy(
                rows0, acc.at[pl.ds(s * ROWS_PER_TILE + k * CHUNK, CHUNK)])
        plsc.subcore_barrier()

        def stage_and_gather(i, b):
            pltpu.sync_copy(eidx_hbm.at[w, :, pl.ds(i * CHUNK, CHUNK)], ix[b])
            pltpu.async_copy(node_hbm.at[ix[b].at[0]], rows[b], gsem[b])

        stage_and_gather(0, 0)
        stage_and_gather(1, 1)

        @pl.loop(0, NFULL // 2)
        def _(t):
            for b in range(2):
                i = 2 * t + b
                pltpu.make_async_copy(node_hbm.at[ix[b].at[0]], rows[b],
                                      gsem[b]).wait()
                pltpu.async_copy(rows[b], acc.at[ix[b].at[1]], ssem[b],
                                 add=True)

                @pl.when(i + 2 < NFULL)
                def _():
                    pltpu.make_async_copy(rows[b], acc.at[ix[b].at[1]],
                                          ssem[b]).wait()
                    stage_and_gather(i + 2, b)

        pltpu.sync_copy(eidx_hbm.at[w, :, pl.ds(NFULL * CHUNK, TAIL)], ixt)
        pltpu.make_async_copy(rows0, acc.at[ix0.at[1]], ssem[0]).wait()
        pltpu.async_copy(node_hbm.at[ixt.at[0]], rows0.at[pl.ds(0, TAIL)], g0)
        pltpu.make_async_copy(node_hbm.at[ixt.at[0]], rows0.at[pl.ds(0, TAIL)],
                              g0).wait()
        pltpu.async_copy(rows0.at[pl.ds(0, TAIL)], acc.at[ixt.at[1]], s0,
                         add=True)
        pltpu.make_async_copy(rows0.at[pl.ds(0, TAIL)], acc.at[ixt.at[1]],
                              s0).wait()
        pltpu.make_async_copy(rows1, acc.at[ix1.at[1]], ssem[1]).wait()

        plsc.subcore_barrier()
        pltpu.sync_copy(
            acc.at[pl.ds(s * ROWS_PER_TILE, ROWS_PER_TILE)],
            out_hbm.at[c, pl.ds(s * ROWS_PER_TILE, ROWS_PER_TILE)])

    return body(node, eidx)


def _combine(partials):
    def body(p_ref, o_ref):
        o_ref[...] = p_ref[0] + p_ref[1]

    return pl.pallas_call(
        body,
        out_shape=jax.ShapeDtypeStruct((NPAD, D), jnp.float32),
    )(partials)


@jax.jit
def kernel(node, edge_index):
    eidx = edge_index.astype(jnp.int32).reshape(2, NW, EPW).transpose(1, 0, 2)
    partials = _sc_partial_sums(node, eidx)
    return _combine(partials)[:N_NODES]
